# Initial kernel scaffold; baseline (speedup 1.0000x reference)
import jax
import jax.numpy as jnp
from jax.experimental import pallas as pl


def kernel(inputs, hidden_state, adj, enc_W, enc_b, w_ih, w_hh, b_ih, b_hh, g1_W, g1_b, g2_W, g2_b, q_W, q_b):
    raise NotImplementedError("write your pallas kernel here")


# speedup vs baseline: 1541.9185x; 1541.9185x over previous
import jax
import jax.numpy as jnp
from jax.experimental import pallas as pl

N = 1024
DIN = 275
H = 64
A = 16


def _fused_body(x_ref, h_ref, adj_ref, encW_ref, encb_ref, wih_ref, whh_ref,
                bih_ref, bhh_ref, g1W_ref, g1b_ref, g2W_ref, g2b_ref,
                qW_ref, qb_ref, q_out_ref, h2_out_ref):
    f32 = jnp.float32

    h1 = jnp.maximum(
        jnp.dot(x_ref[...], encW_ref[...], preferred_element_type=f32)
        + encb_ref[...], 0.0)

    h = h_ref[...]
    gi = jax.lax.dot_general(h1, wih_ref[...], (((1,), (1,)), ((), ())),
                             preferred_element_type=f32) + bih_ref[...]
    gh = jax.lax.dot_general(h, whh_ref[...], (((1,), (1,)), ((), ())),
                             preferred_element_type=f32) + bhh_ref[...]
    r = jax.nn.sigmoid(gi[:, :H] + gh[:, :H])
    z = jax.nn.sigmoid(gi[:, H:2 * H] + gh[:, H:2 * H])
    n = jnp.tanh(gi[:, 2 * H:] + r * gh[:, 2 * H:])
    h2 = (1.0 - z) * n + z * h
    h2_out_ref[...] = h2

    adj01 = jnp.where(adj_ref[...] != 0, 1.0, 0.0).astype(f32)
    deg = 1.0 + jnp.sum(adj01, axis=0, keepdims=True)
    dinv_col = jax.lax.rsqrt(deg).reshape(N, 1)

    u1 = dinv_col * jnp.dot(h2, g1W_ref[...], preferred_element_type=f32)
    agg1 = jax.lax.dot_general(adj01, u1, (((0,), (0,)), ((), ())),
                               preferred_element_type=f32)
    h3 = jnp.maximum(dinv_col * (agg1 + u1) + g1b_ref[...], 0.0)

    u2 = dinv_col * jnp.dot(h3, g2W_ref[...], preferred_element_type=f32)
    agg2 = jax.lax.dot_general(adj01, u2, (((0,), (0,)), ((), ())),
                               preferred_element_type=f32)
    h4 = dinv_col * (agg2 + u2) + g2b_ref[...]

    q_out_ref[...] = (jnp.dot(h4, qW_ref[...], preferred_element_type=f32)
                      + qb_ref[...])


def kernel(inputs, hidden_state, adj, enc_W, enc_b, w_ih, w_hh, b_ih, b_hh,
           g1_W, g1_b, g2_W, g2_b, q_W, q_b):
    hidden_state = hidden_state.reshape(N, H)
    out = pl.pallas_call(
        _fused_body,
        out_shape=(jax.ShapeDtypeStruct((N, A), jnp.float32),
                   jax.ShapeDtypeStruct((N, H), jnp.float32)),
    )(inputs, hidden_state, adj, enc_W, enc_b.reshape(1, H),
      w_ih, w_hh, b_ih.reshape(1, 3 * H), b_hh.reshape(1, 3 * H),
      g1_W, g1_b.reshape(1, H), g2_W, g2_b.reshape(1, H),
      q_W, q_b.reshape(1, A))
    return out
